# baseline (speedup 1.0000x reference)
import functools

import jax
import jax.numpy as jnp
from jax import lax
from jax.experimental import pallas as pl
from jax.experimental.pallas import tpu as pltpu
from jax.experimental.pallas import tpu_sc as plsc

DISC_SIZE = 50
TAB_DIM = DISC_SIZE + 1
TAB_FLAT = TAB_DIM * TAB_DIM
TAB_ROWS = 32

N_POINTS = 1048576
NUM_WORKERS = 32
PER_W = N_POINTS // NUM_WORKERS
IN_ROWS = 2 * N_POINTS // 128
OUT_ROWS = N_POINTS // 128
IN_ROWS_W = IN_ROWS // NUM_WORKERS
OUT_ROWS_W = OUT_ROWS // NUM_WORKERS
VECS = PER_W // 16


BLOCKS = N_POINTS // 128
BLOCKS_W = BLOCKS // NUM_WORKERS


NCHUNK = 8
CHUNK_B = BLOCKS_W // NCHUNK


@functools.partial(
    pl.kernel,
    out_type=jax.ShapeDtypeStruct((OUT_ROWS, 128), jnp.float32),
    mesh=plsc.VectorSubcoreMesh(core_axis_name="c", subcore_axis_name="s"),
    compiler_params=pltpu.CompilerParams(needs_layout_passes=False),
    scratch_types=[
        pltpu.VMEM((2, CHUNK_B, 2, 128), jnp.float32),
        pltpu.VMEM((TAB_ROWS * 128,), jnp.float32),
        pltpu.VMEM((2, CHUNK_B, 128), jnp.float32),
        pltpu.SemaphoreType.DMA,
        pltpu.SemaphoreType.DMA,
        pltpu.SemaphoreType.DMA,
        pltpu.SemaphoreType.DMA,
    ],
)
def _disc_count_kernel(ob_hbm, tab_hbm, out_hbm, ob_v, tab_v, out_v,
                       sem_i0, sem_i1, sem_o0, sem_o1):
    wid = lax.axis_index("s") * 2 + lax.axis_index("c")
    base = wid * BLOCKS_W
    sems_i = (sem_i0, sem_i1)
    sems_o = (sem_o0, sem_o1)

    def in_copy(c):
        return pltpu.async_copy(
            ob_hbm.at[pl.ds(base + c * CHUNK_B, CHUNK_B)],
            ob_v.at[c % 2], sems_i[c % 2])

    cps_in = {0: in_copy(0)}
    cps_out = {}

    pltpu.sync_copy(tab_hbm, tab_v)

    def inv_body(i, _):
        for j in range(4):
            o = i * 64 + j * 16
            tab_v[pl.ds(o, 16)] = 1.0 / tab_v[pl.ds(o, 16)]
        return 0

    lax.fori_loop(0, TAB_ROWS * 2, inv_body, 0)

    inv_factor = jnp.float32(1.0 / DISC_SIZE)

    for c in range(NCHUNK):
        buf = c % 2
        if c + 1 < NCHUNK:
            cps_in[c + 1] = in_copy(c + 1)
        cps_in[c].wait()
        if c >= 2:
            cps_out[c - 2].wait()

        @plsc.parallel_loop(0, CHUNK_B, unroll=2)
        def body(b, buf=buf):
            xs = [ob_v[buf, b, 0, pl.ds(j * 16, 16)] for j in range(8)]
            ys = [ob_v[buf, b, 1, pl.ds(j * 16, 16)] for j in range(8)]
            fis = [(x / inv_factor).astype(jnp.int32) * TAB_DIM
                   + (y / inv_factor).astype(jnp.int32)
                   for x, y in zip(xs, ys)]
            rvs = [plsc.load_gather(tab_v, [fi]) for fi in fis]
            for j in range(8):
                out_v[buf, b, pl.ds(j * 16, 16)] = rvs[j]
        cps_out[c] = pltpu.async_copy(
            out_v.at[buf],
            out_hbm.at[pl.ds(wid * OUT_ROWS_W + c * CHUNK_B, CHUNK_B)],
            sems_o[buf])

    cps_out[NCHUNK - 2].wait()
    cps_out[NCHUNK - 1].wait()


def kernel(ob_no, count_table):
    ob3d = ob_no.reshape(BLOCKS, 128, 2).transpose(0, 2, 1)
    tab_flat = jnp.concatenate(
        [count_table.reshape(-1),
         jnp.ones((TAB_ROWS * 128 - TAB_FLAT,), jnp.float32)])
    out2d = _disc_count_kernel(ob3d, tab_flat)
    return out2d.reshape(N_POINTS)


# speedup vs baseline: 1.0815x; 1.0815x over previous
import functools

import jax
import jax.numpy as jnp
from jax import lax
from jax.experimental import pallas as pl
from jax.experimental.pallas import tpu as pltpu
from jax.experimental.pallas import tpu_sc as plsc

DISC_SIZE = 50
TAB_DIM = DISC_SIZE + 1
TAB_FLAT = TAB_DIM * TAB_DIM
TAB_ROWS = 32

N_POINTS = 1048576
NUM_WORKERS = 32
PER_W = N_POINTS // NUM_WORKERS
IN_ROWS = 2 * N_POINTS // 128
OUT_ROWS = N_POINTS // 128
IN_ROWS_W = IN_ROWS // NUM_WORKERS
OUT_ROWS_W = OUT_ROWS // NUM_WORKERS
VECS = PER_W // 16


BLOCKS = N_POINTS // 128
BLOCKS_W = BLOCKS // NUM_WORKERS


NCHUNK = 4
CHUNK_B = BLOCKS_W // NCHUNK


@functools.partial(
    pl.kernel,
    out_type=jax.ShapeDtypeStruct((OUT_ROWS, 128), jnp.float32),
    mesh=plsc.VectorSubcoreMesh(core_axis_name="c", subcore_axis_name="s"),
    compiler_params=pltpu.CompilerParams(needs_layout_passes=False),
    scratch_types=[
        pltpu.VMEM((2, CHUNK_B, 2, 128), jnp.float32),
        pltpu.VMEM((TAB_ROWS * 128,), jnp.float32),
        pltpu.VMEM((2, CHUNK_B, 128), jnp.float32),
        pltpu.SemaphoreType.DMA,
        pltpu.SemaphoreType.DMA,
        pltpu.SemaphoreType.DMA,
        pltpu.SemaphoreType.DMA,
    ],
)
def _disc_count_kernel(ob_hbm, tab_hbm, out_hbm, ob_v, tab_v, out_v,
                       sem_i0, sem_i1, sem_o0, sem_o1):
    wid = lax.axis_index("s") * 2 + lax.axis_index("c")
    base = wid * BLOCKS_W
    sems_i = (sem_i0, sem_i1)
    sems_o = (sem_o0, sem_o1)

    def in_copy(c):
        return pltpu.async_copy(
            ob_hbm.at[pl.ds(base + c * CHUNK_B, CHUNK_B)],
            ob_v.at[c % 2], sems_i[c % 2])

    cps_in = {0: in_copy(0)}
    cps_out = {}

    pltpu.sync_copy(tab_hbm, tab_v)

    def inv_body(i, _):
        for j in range(4):
            o = i * 64 + j * 16
            tab_v[pl.ds(o, 16)] = 1.0 / tab_v[pl.ds(o, 16)]
        return 0

    lax.fori_loop(0, TAB_ROWS * 2, inv_body, 0)

    inv_factor = jnp.float32(1.0 / DISC_SIZE)

    for c in range(NCHUNK):
        buf = c % 2
        if c + 1 < NCHUNK:
            cps_in[c + 1] = in_copy(c + 1)
        cps_in[c].wait()
        if c >= 2:
            cps_out[c - 2].wait()

        @plsc.parallel_loop(0, CHUNK_B, unroll=2)
        def body(b, buf=buf):
            xs = [ob_v[buf, b, 0, pl.ds(j * 16, 16)] for j in range(8)]
            ys = [ob_v[buf, b, 1, pl.ds(j * 16, 16)] for j in range(8)]
            fis = [(x / inv_factor).astype(jnp.int32) * TAB_DIM
                   + (y / inv_factor).astype(jnp.int32)
                   for x, y in zip(xs, ys)]
            rvs = [plsc.load_gather(tab_v, [fi]) for fi in fis]
            for j in range(8):
                out_v[buf, b, pl.ds(j * 16, 16)] = rvs[j]
        cps_out[c] = pltpu.async_copy(
            out_v.at[buf],
            out_hbm.at[pl.ds(wid * OUT_ROWS_W + c * CHUNK_B, CHUNK_B)],
            sems_o[buf])

    cps_out[NCHUNK - 2].wait()
    cps_out[NCHUNK - 1].wait()


def kernel(ob_no, count_table):
    ob3d = ob_no.reshape(BLOCKS, 128, 2).transpose(0, 2, 1)
    tab_flat = jnp.concatenate(
        [count_table.reshape(-1),
         jnp.ones((TAB_ROWS * 128 - TAB_FLAT,), jnp.float32)])
    out2d = _disc_count_kernel(ob3d, tab_flat)
    return out2d.reshape(N_POINTS)
